# baseline (speedup 1.0000x reference)
import jax
import jax.numpy as jnp
from jax.experimental import pallas as pl

_TN = 256


def _rvq_body(x_ref, cb_ref, e2_ref, qout_ref, idx_ref, loss_ref):
    step = pl.program_id(0)
    r = x_ref[...]
    nq = cb_ref.shape[0]
    qout = jnp.zeros_like(r)
    idx_cols = []
    loss_vals = []
    for q in range(nq):
        embed = cb_ref[q]
        r2 = jnp.sum(r * r, axis=1, keepdims=True)
        scores = jax.lax.dot_general(
            r.astype(jnp.bfloat16), embed.astype(jnp.bfloat16),
            (((1,), (1,)), ((), ())),
            preferred_element_type=jnp.float32)
        dist = (r2 - 2.0 * scores) + e2_ref[q:q + 1, :]
        idx = jnp.argmin(dist, axis=1, keepdims=True).astype(jnp.int32)
        onehot = (jax.lax.broadcasted_iota(jnp.int32, dist.shape, 1)
                  == idx).astype(jnp.float32)
        quant = jax.lax.dot_general(
            onehot, embed, (((1,), (0,)), ((), ())),
            precision=jax.lax.Precision.HIGHEST,
            preferred_element_type=jnp.float32)
        diff = quant - r
        loss_vals.append(jnp.sum(diff * diff))
        idx_cols.append(idx)
        qout = qout + quant
        r = r - quant
    qout_ref[...] = qout
    idx_ref[...] = jnp.concatenate(idx_cols, axis=1)
    losses = jnp.stack(loss_vals).reshape(1, nq)

    @pl.when(step == 0)
    def _init():
        loss_ref[...] = jnp.zeros_like(loss_ref)

    loss_ref[...] = loss_ref[...] + losses


def kernel(x, codebooks):
    B, N, D = x.shape
    NQ, K, _ = codebooks.shape
    T = B * N
    xf = x.reshape(T, D)
    e2 = jnp.sum(codebooks * codebooks, axis=-1)
    qout, idx, loss = pl.pallas_call(
        _rvq_body,
        grid=(T // _TN,),
        in_specs=[
            pl.BlockSpec((_TN, D), lambda i: (i, 0)),
            pl.BlockSpec((NQ, K, D), lambda i: (0, 0, 0)),
            pl.BlockSpec((NQ, K), lambda i: (0, 0)),
        ],
        out_specs=[
            pl.BlockSpec((_TN, D), lambda i: (i, 0)),
            pl.BlockSpec((_TN, NQ), lambda i: (i, 0)),
            pl.BlockSpec((1, NQ), lambda i: (0, 0)),
        ],
        out_shape=[
            jax.ShapeDtypeStruct((T, D), jnp.float32),
            jax.ShapeDtypeStruct((T, NQ), jnp.int32),
            jax.ShapeDtypeStruct((1, NQ), jnp.float32),
        ],
    )(xf, codebooks, e2)
    quantized_out = qout.reshape(B, N, D)
    indices = idx.reshape(B, N, NQ)
    losses = loss[0] / (B * N * D)
    return quantized_out, indices, losses


# speedup vs baseline: 1.0087x; 1.0087x over previous
import jax
import jax.numpy as jnp
from jax.experimental import pallas as pl

_TN = 256


def _rvq_body(x_ref, cb_ref, cbt_ref, e2_ref, qout_ref, idx_ref, loss_ref):
    step = pl.program_id(0)
    r = x_ref[...]
    nq = cb_ref.shape[0]
    qout = jnp.zeros_like(r)
    idx_cols = []
    loss_vals = []
    for q in range(nq):
        embed = cb_ref[q]
        embed_t = cbt_ref[q]
        r2 = jnp.sum(r * r, axis=1, keepdims=True)
        scores = jax.lax.dot_general(
            r.astype(jnp.bfloat16), embed_t,
            (((1,), (0,)), ((), ())),
            preferred_element_type=jnp.float32)
        dist = (r2 - 2.0 * scores) + e2_ref[q:q + 1, :]
        idx = jnp.argmin(dist, axis=1, keepdims=True).astype(jnp.int32)
        onehot = (jax.lax.broadcasted_iota(jnp.int32, dist.shape, 1)
                  == idx).astype(jnp.float32)
        quant = jax.lax.dot_general(
            onehot, embed, (((1,), (0,)), ((), ())),
            precision=jax.lax.Precision.HIGHEST,
            preferred_element_type=jnp.float32)
        diff = quant - r
        loss_vals.append(jnp.sum(diff * diff))
        idx_cols.append(idx)
        qout = qout + quant
        r = r - quant
    qout_ref[...] = qout
    idx_ref[...] = jnp.concatenate(idx_cols, axis=1)
    losses = jnp.stack(loss_vals).reshape(1, nq)

    @pl.when(step == 0)
    def _init():
        loss_ref[...] = jnp.zeros_like(loss_ref)

    loss_ref[...] = loss_ref[...] + losses


def kernel(x, codebooks):
    B, N, D = x.shape
    NQ, K, _ = codebooks.shape
    T = B * N
    xf = x.reshape(T, D)
    e2 = jnp.sum(codebooks * codebooks, axis=-1)
    cbt = jnp.swapaxes(codebooks, 1, 2).astype(jnp.bfloat16)
    qout, idx, loss = pl.pallas_call(
        _rvq_body,
        grid=(T // _TN,),
        in_specs=[
            pl.BlockSpec((_TN, D), lambda i: (i, 0)),
            pl.BlockSpec((NQ, K, D), lambda i: (0, 0, 0)),
            pl.BlockSpec((NQ, D, K), lambda i: (0, 0, 0)),
            pl.BlockSpec((NQ, K), lambda i: (0, 0)),
        ],
        out_specs=[
            pl.BlockSpec((_TN, D), lambda i: (i, 0)),
            pl.BlockSpec((_TN, NQ), lambda i: (i, 0)),
            pl.BlockSpec((1, NQ), lambda i: (0, 0)),
        ],
        out_shape=[
            jax.ShapeDtypeStruct((T, D), jnp.float32),
            jax.ShapeDtypeStruct((T, NQ), jnp.int32),
            jax.ShapeDtypeStruct((1, NQ), jnp.float32),
        ],
    )(xf, codebooks, cbt, e2)
    quantized_out = qout.reshape(B, N, D)
    indices = idx.reshape(B, N, NQ)
    losses = loss[0] / (B * N * D)
    return quantized_out, indices, losses


# speedup vs baseline: 5.3964x; 5.3498x over previous
import jax
import jax.numpy as jnp
from jax.experimental import pallas as pl

_TN = 256
_CHUNK = 128


def _rvq_body(x_ref, cbt_ref, cbt32_ref, e2_ref, qout_ref, idx_ref, loss_ref):
    step = pl.program_id(0)
    r = x_ref[...]
    nq, _, k = cbt_ref.shape
    tn, d = r.shape
    n_chunks = k // _CHUNK
    qout = jnp.zeros_like(r)
    idx_cols = []
    loss_vals = []
    for q in range(nq):
        embed_t = cbt_ref[q]
        embed_t32 = cbt32_ref[q]
        r2 = jnp.sum(r * r, axis=1, keepdims=True)
        scores = jax.lax.dot_general(
            r.astype(jnp.bfloat16), embed_t,
            (((1,), (0,)), ((), ())),
            preferred_element_type=jnp.float32)
        dist = (r2 - 2.0 * scores) + e2_ref[q:q + 1, :]
        idx = jnp.argmin(dist, axis=1, keepdims=True).astype(jnp.int32)
        idx_l = jnp.swapaxes(idx, 0, 1)
        a_star = jnp.broadcast_to(idx_l & (_CHUNK - 1), (d, tn))
        c_star = jnp.broadcast_to(idx_l >> 7, (d, tn))
        quant_t = jnp.zeros((d, tn), dtype=jnp.float32)
        for c in range(n_chunks):
            src = embed_t32[:, c * _CHUNK:(c + 1) * _CHUNK]
            sel = jnp.take_along_axis(src, a_star, axis=1)
            quant_t = jnp.where(c_star == c, sel, quant_t)
        quant = jnp.swapaxes(quant_t, 0, 1)
        diff = quant - r
        loss_vals.append(jnp.sum(diff * diff))
        idx_cols.append(idx)
        qout = qout + quant
        r = r - quant
    qout_ref[...] = qout
    idx_ref[...] = jnp.concatenate(idx_cols, axis=1)
    losses = jnp.stack(loss_vals).reshape(1, nq)

    @pl.when(step == 0)
    def _init():
        loss_ref[...] = jnp.zeros_like(loss_ref)

    loss_ref[...] = loss_ref[...] + losses


def kernel(x, codebooks):
    B, N, D = x.shape
    NQ, K, _ = codebooks.shape
    T = B * N
    xf = x.reshape(T, D)
    e2 = jnp.sum(codebooks * codebooks, axis=-1)
    cbt32 = jnp.swapaxes(codebooks, 1, 2)
    cbt = cbt32.astype(jnp.bfloat16)
    qout, idx, loss = pl.pallas_call(
        _rvq_body,
        grid=(T // _TN,),
        in_specs=[
            pl.BlockSpec((_TN, D), lambda i: (i, 0)),
            pl.BlockSpec((NQ, D, K), lambda i: (0, 0, 0)),
            pl.BlockSpec((NQ, D, K), lambda i: (0, 0, 0)),
            pl.BlockSpec((NQ, K), lambda i: (0, 0)),
        ],
        out_specs=[
            pl.BlockSpec((_TN, D), lambda i: (i, 0)),
            pl.BlockSpec((_TN, NQ), lambda i: (i, 0)),
            pl.BlockSpec((1, NQ), lambda i: (0, 0)),
        ],
        out_shape=[
            jax.ShapeDtypeStruct((T, D), jnp.float32),
            jax.ShapeDtypeStruct((T, NQ), jnp.int32),
            jax.ShapeDtypeStruct((1, NQ), jnp.float32),
        ],
    )(xf, cbt, cbt32, e2)
    quantized_out = qout.reshape(B, N, D)
    indices = idx.reshape(B, N, NQ)
    losses = loss[0] / (B * N * D)
    return quantized_out, indices, losses


# speedup vs baseline: 5.8239x; 1.0792x over previous
import jax
import jax.numpy as jnp
from jax.experimental import pallas as pl

_TN = 512
_CHUNK = 128


def _rvq_body(x_ref, cbt_ref, cbt32_ref, e2_ref, qout_ref, idx_ref, loss_ref):
    step = pl.program_id(0)
    r = x_ref[...]
    nq, _, k = cbt_ref.shape
    tn, d = r.shape
    n_chunks = k // _CHUNK
    qout = jnp.zeros_like(r)
    idx_cols = []
    loss_vals = []
    for q in range(nq):
        embed_t = cbt_ref[q]
        embed_t32 = cbt32_ref[q]
        r2 = jnp.sum(r * r, axis=1, keepdims=True)
        scores = jax.lax.dot_general(
            r.astype(jnp.bfloat16), embed_t,
            (((1,), (0,)), ((), ())),
            preferred_element_type=jnp.float32)
        dist = (r2 - 2.0 * scores) + e2_ref[q:q + 1, :]
        idx = jnp.argmin(dist, axis=1, keepdims=True).astype(jnp.int32)
        idx_l = jnp.swapaxes(idx, 0, 1)
        a_star = jnp.broadcast_to(idx_l & (_CHUNK - 1), (d, tn))
        c_star = jnp.broadcast_to(idx_l >> 7, (d, tn))
        quant_t = jnp.zeros((d, tn), dtype=jnp.float32)
        for c in range(n_chunks):
            src = embed_t32[:, c * _CHUNK:(c + 1) * _CHUNK]
            sel = jnp.take_along_axis(src, a_star, axis=1)
            quant_t = jnp.where(c_star == c, sel, quant_t)
        quant = jnp.swapaxes(quant_t, 0, 1)
        diff = quant - r
        loss_vals.append(jnp.sum(diff * diff))
        idx_cols.append(idx)
        qout = qout + quant
        r = r - quant
    qout_ref[...] = qout
    idx_ref[...] = jnp.concatenate(idx_cols, axis=1)
    losses = jnp.stack(loss_vals).reshape(1, nq)

    @pl.when(step == 0)
    def _init():
        loss_ref[...] = jnp.zeros_like(loss_ref)

    loss_ref[...] = loss_ref[...] + losses


def kernel(x, codebooks):
    B, N, D = x.shape
    NQ, K, _ = codebooks.shape
    T = B * N
    xf = x.reshape(T, D)
    e2 = jnp.sum(codebooks * codebooks, axis=-1)
    cbt32 = jnp.swapaxes(codebooks, 1, 2)
    cbt = cbt32.astype(jnp.bfloat16)
    qout, idx, loss = pl.pallas_call(
        _rvq_body,
        grid=(T // _TN,),
        in_specs=[
            pl.BlockSpec((_TN, D), lambda i: (i, 0)),
            pl.BlockSpec((NQ, D, K), lambda i: (0, 0, 0)),
            pl.BlockSpec((NQ, D, K), lambda i: (0, 0, 0)),
            pl.BlockSpec((NQ, K), lambda i: (0, 0)),
        ],
        out_specs=[
            pl.BlockSpec((_TN, D), lambda i: (i, 0)),
            pl.BlockSpec((_TN, NQ), lambda i: (i, 0)),
            pl.BlockSpec((1, NQ), lambda i: (0, 0)),
        ],
        out_shape=[
            jax.ShapeDtypeStruct((T, D), jnp.float32),
            jax.ShapeDtypeStruct((T, NQ), jnp.int32),
            jax.ShapeDtypeStruct((1, NQ), jnp.float32),
        ],
    )(xf, cbt, cbt32, e2)
    quantized_out = qout.reshape(B, N, D)
    indices = idx.reshape(B, N, NQ)
    losses = loss[0] / (B * N * D)
    return quantized_out, indices, losses
